# baseline (speedup 1.0000x reference)
import functools

import jax
import jax.numpy as jnp
from jax import lax
from jax.experimental import pallas as pl
from jax.experimental.pallas import tpu as pltpu
from jax.experimental.pallas import tpu_sc as plsc

VOCAB = 1000000
EMB = 64
NUM_L1 = 32
NUM_LEAF = 1024
B = 16384
L = 200
SPLIT = 104

NC, NS = 2, 16
NW = NC * NS
ROWS_PER_W = B // NW
PB = 4
N_STEP = ROWS_PER_W // PB
LANES = 16
C_CHUNKS = EMB // LANES
UNROLL = 8


def _gather_sum_kernel(x_hbm, emb_hbm, out_hbm,
                       idx0, idx1, rows0, rows1, os0, os1,
                       isem0, isem1, gsem0, gsem1, osem0, osem1):
    idx_v = (idx0, idx1)
    rows_v = (rows0, rows1)
    out_v = (os0, os1)
    isem = (isem0, isem1)
    gsem = (gsem0, gsem1)
    osem = (osem0, osem1)
    wid = lax.axis_index("s") * NC + lax.axis_index("c")
    base = wid * ROWS_PER_W

    def idx_start(slot, step):
        pltpu.async_copy(
            x_hbm.at[pl.ds(base + step * PB, PB)], idx_v[slot], isem[slot])

    def idx_wait(slot):
        pltpu.make_async_copy(
            x_hbm.at[pl.ds(0, PB)], idx_v[slot], isem[slot]).wait()

    def gathers_start(slot):
        for k in range(PB):
            pltpu.async_copy(
                emb_hbm.at[idx_v[slot].at[k, pl.ds(0, SPLIT)]],
                rows_v[slot].at[pl.ds(k * L, SPLIT)],
                gsem[slot])
            pltpu.async_copy(
                emb_hbm.at[idx_v[slot].at[k, pl.ds(SPLIT, L - SPLIT)]],
                rows_v[slot].at[pl.ds(k * L + SPLIT, L - SPLIT)],
                gsem[slot])

    def gathers_wait(slot):
        pltpu.make_async_copy(
            emb_hbm.at[pl.ds(0, PB * L)], rows_v[slot], gsem[slot]).wait()

    def out_start(slot, step):
        pltpu.async_copy(
            out_v[slot], out_hbm.at[pl.ds(base + step * PB, PB)], osem[slot])

    def out_wait(slot):
        pltpu.make_async_copy(
            out_v[slot], out_hbm.at[pl.ds(0, PB)], osem[slot]).wait()

    def count_recips(slot):
        lane = lax.iota(jnp.int32, 16)
        recips = []
        for k in range(PB):
            total = jnp.zeros((LANES,), jnp.int32)
            for c in range(12):
                ch = idx_v[slot][k, pl.ds(c * LANES, LANES)]
                total = total + plsc.all_reduce_population_count(ch != 0)
            tail = idx_v[slot][k, pl.ds(L - LANES, LANES)]
            total = total + plsc.all_reduce_population_count(
                (tail != 0) & (lane >= 12 * LANES - (L - LANES)))
            recips.append(1.0 / total.astype(jnp.float32))
        return recips

    def reduce(slot, recips):
        zero = jnp.zeros((LANES,), jnp.float32)
        for k in range(PB):
            rbase = k * L

            def body(r, acc, rbase=rbase, slot=slot):
                a = list(acc)
                for u in range(UNROLL):
                    rr = rbase + r * UNROLL + u
                    for c in range(C_CHUNKS):
                        a[c] = a[c] + rows_v[slot][rr, pl.ds(c * LANES, LANES)]
                return tuple(a)

            acc = lax.fori_loop(0, L // UNROLL, body, (zero,) * C_CHUNKS)
            for c in range(C_CHUNKS):
                out_v[slot][k, pl.ds(c * LANES, LANES)] = (
                    jnp.maximum(acc[c] * recips[k], 0.0))

    idx_start(0, 0)
    idx_wait(0)
    gathers_start(0)
    idx_start(1, 1)

    def body(i2, _):
        s0 = 2 * i2
        s1 = s0 + 1
        idx_wait(1)
        gathers_start(1)
        gathers_wait(0)
        r0 = count_recips(0)

        @pl.when(s0 + 2 < N_STEP)
        def _():
            idx_start(0, s0 + 2)

        @pl.when(s0 >= 2)
        def _():
            out_wait(0)

        reduce(0, r0)
        out_start(0, s0)

        @pl.when(s0 + 2 < N_STEP)
        def _():
            idx_wait(0)
            gathers_start(0)

        gathers_wait(1)
        r1 = count_recips(1)

        @pl.when(s1 + 2 < N_STEP)
        def _():
            idx_start(1, s1 + 2)

        @pl.when(s1 >= 2)
        def _():
            out_wait(1)

        reduce(1, r1)
        out_start(1, s1)
        return 0

    lax.fori_loop(0, N_STEP // 2, body, 0)
    out_wait(0)
    out_wait(1)


_gather_sum = functools.partial(
    pl.kernel,
    out_type=jax.ShapeDtypeStruct((B, EMB), jnp.float32),
    mesh=plsc.VectorSubcoreMesh(core_axis_name="c", subcore_axis_name="s"),
    scratch_types=[
        pltpu.VMEM((PB, L), jnp.int32),
        pltpu.VMEM((PB, L), jnp.int32),
        pltpu.VMEM((PB * L, EMB), jnp.float32),
        pltpu.VMEM((PB * L, EMB), jnp.float32),
        pltpu.VMEM((PB, EMB), jnp.float32),
        pltpu.VMEM((PB, EMB), jnp.float32),
        pltpu.SemaphoreType.DMA,
        pltpu.SemaphoreType.DMA,
        pltpu.SemaphoreType.DMA,
        pltpu.SemaphoreType.DMA,
        pltpu.SemaphoreType.DMA,
        pltpu.SemaphoreType.DMA,
    ],
    compiler_params=pltpu.CompilerParams(
        use_tc_tiling_on_sc=False, needs_layout_passes=False),
)(_gather_sum_kernel)


CBLK = 2000


def _compact_body(in_ref, out_ref):
    e = in_ref[...].reshape(CBLK // 2, 2, EMB)
    out_ref[...] = jnp.concatenate([e[:, 0, :], e[:, 1, :]], axis=1)


def _compact_table(emb):
    return pl.pallas_call(
        _compact_body,
        grid=(VOCAB // CBLK,),
        in_specs=[pl.BlockSpec((CBLK, EMB), lambda i: (i, 0))],
        out_specs=pl.BlockSpec((CBLK // 2, 2 * EMB), lambda i: (i, 0)),
        out_shape=jax.ShapeDtypeStruct((VOCAB // 2, 2 * EMB), jnp.float32),
    )(emb)


BLK = 2048


def _dense_body(h_ref, lab_ref, w1_ref, b1_ref, w2_ref, b2_ref,
                l1_ref, leaf_ref):
    h = h_ref[...]
    l1_ref[...] = (
        jnp.dot(h, w1_ref[...], preferred_element_type=jnp.float32)
        + b1_ref[...]
    )
    one_hot = (
        lab_ref[...]
        == lax.broadcasted_iota(jnp.int32, (BLK, NUM_L1), 1)
    ).astype(jnp.float32)
    leaf_ref[...] = (
        jnp.dot(h, w2_ref[0:EMB, :], preferred_element_type=jnp.float32)
        + jnp.dot(one_hot, w2_ref[EMB:, :], preferred_element_type=jnp.float32)
        + b2_ref[...]
    )


def kernel(x, level1_labels, emb, W1, b1, W2, b2):
    h = _gather_sum(x, _compact_table(emb).reshape(VOCAB, EMB))

    lab2d = level1_labels.reshape(B, 1)
    grid = B // BLK
    l1, leaf = pl.pallas_call(
        _dense_body,
        grid=(grid,),
        in_specs=[
            pl.BlockSpec((BLK, EMB), lambda i: (i, 0)),
            pl.BlockSpec((BLK, 1), lambda i: (i, 0)),
            pl.BlockSpec((EMB, NUM_L1), lambda i: (0, 0)),
            pl.BlockSpec((1, NUM_L1), lambda i: (0, 0)),
            pl.BlockSpec((EMB + NUM_L1, NUM_LEAF), lambda i: (0, 0)),
            pl.BlockSpec((1, NUM_LEAF), lambda i: (0, 0)),
        ],
        out_specs=[
            pl.BlockSpec((BLK, NUM_L1), lambda i: (i, 0)),
            pl.BlockSpec((BLK, NUM_LEAF), lambda i: (i, 0)),
        ],
        out_shape=[
            jax.ShapeDtypeStruct((B, NUM_L1), jnp.float32),
            jax.ShapeDtypeStruct((B, NUM_LEAF), jnp.float32),
        ],
    )(h, lab2d, W1, b1.reshape(1, NUM_L1), W2, b2.reshape(1, NUM_LEAF))
    return (l1, leaf)


# speedup vs baseline: 1.3110x; 1.3110x over previous
import functools

import jax
import jax.numpy as jnp
from jax import lax
from jax.experimental import pallas as pl
from jax.experimental.pallas import tpu as pltpu
from jax.experimental.pallas import tpu_sc as plsc

VOCAB = 1000000
EMB = 64
NUM_L1 = 32
NUM_LEAF = 1024
B = 16384
L = 200
SPLIT = 104

NC, NS = 2, 16
NW = NC * NS
ROWS_PER_W = B // NW
PB = 4
N_STEP = ROWS_PER_W // PB
LANES = 16
C_CHUNKS = EMB // LANES
UNROLL = 8


def _gather_sum_kernel(x_hbm, emb_hbm, out_hbm,
                       idx0, idx1, rows0, rows1, os0, os1,
                       isem0, isem1, gsem0, gsem1, osem0, osem1):
    idx_v = (idx0, idx1)
    rows_v = (rows0, rows1)
    out_v = (os0, os1)
    isem = (isem0, isem1)
    gsem = (gsem0, gsem1)
    osem = (osem0, osem1)
    wid = lax.axis_index("s") * NC + lax.axis_index("c")
    base = wid * ROWS_PER_W

    def idx_start(slot, step):
        pltpu.async_copy(
            x_hbm.at[pl.ds(base + step * PB, PB)], idx_v[slot], isem[slot])

    def idx_wait(slot):
        pltpu.make_async_copy(
            x_hbm.at[pl.ds(0, PB)], idx_v[slot], isem[slot]).wait()

    def gathers_start(slot):
        for k in range(PB):
            pltpu.async_copy(
                emb_hbm.at[idx_v[slot].at[k, pl.ds(0, SPLIT)]],
                rows_v[slot].at[pl.ds(k * L, SPLIT)],
                gsem[slot])
            pltpu.async_copy(
                emb_hbm.at[idx_v[slot].at[k, pl.ds(SPLIT, L - SPLIT)]],
                rows_v[slot].at[pl.ds(k * L + SPLIT, L - SPLIT)],
                gsem[slot])

    def gathers_wait(slot):
        pltpu.make_async_copy(
            emb_hbm.at[pl.ds(0, PB * L)], rows_v[slot], gsem[slot]).wait()

    def out_start(slot, step):
        pltpu.async_copy(
            out_v[slot], out_hbm.at[pl.ds(base + step * PB, PB)], osem[slot])

    def out_wait(slot):
        pltpu.make_async_copy(
            out_v[slot], out_hbm.at[pl.ds(0, PB)], osem[slot]).wait()

    def count_recips(slot):
        lane = lax.iota(jnp.int32, 16)
        recips = []
        for k in range(PB):
            total = jnp.zeros((LANES,), jnp.int32)
            for c in range(12):
                ch = idx_v[slot][k, pl.ds(c * LANES, LANES)]
                total = total + plsc.all_reduce_population_count(ch != 0)
            tail = idx_v[slot][k, pl.ds(L - LANES, LANES)]
            total = total + plsc.all_reduce_population_count(
                (tail != 0) & (lane >= 12 * LANES - (L - LANES)))
            recips.append(1.0 / total.astype(jnp.float32))
        return recips

    def reduce(slot, recips):
        zero = jnp.zeros((LANES,), jnp.float32)
        for k in range(PB):
            rbase = k * L

            def body(r, acc, rbase=rbase, slot=slot):
                a = list(acc)
                for u in range(UNROLL):
                    rr = rbase + r * UNROLL + u
                    for c in range(C_CHUNKS):
                        a[c] = a[c] + rows_v[slot][rr, pl.ds(c * LANES, LANES)]
                return tuple(a)

            acc = lax.fori_loop(0, L // UNROLL, body, (zero,) * C_CHUNKS)
            for c in range(C_CHUNKS):
                out_v[slot][k, pl.ds(c * LANES, LANES)] = (
                    jnp.maximum(acc[c] * recips[k], 0.0))

    idx_start(0, 0)
    idx_wait(0)
    gathers_start(0)
    idx_start(1, 1)

    def body(i2, _):
        s0 = 2 * i2
        s1 = s0 + 1
        idx_wait(1)
        gathers_start(1)
        gathers_wait(0)
        r0 = count_recips(0)

        @pl.when(s0 + 2 < N_STEP)
        def _():
            idx_start(0, s0 + 2)

        @pl.when(s0 >= 2)
        def _():
            out_wait(0)

        reduce(0, r0)
        out_start(0, s0)

        @pl.when(s0 + 2 < N_STEP)
        def _():
            idx_wait(0)
            gathers_start(0)

        gathers_wait(1)
        r1 = count_recips(1)

        @pl.when(s1 + 2 < N_STEP)
        def _():
            idx_start(1, s1 + 2)

        @pl.when(s1 >= 2)
        def _():
            out_wait(1)

        reduce(1, r1)
        out_start(1, s1)
        return 0

    lax.fori_loop(0, N_STEP // 2, body, 0)
    out_wait(0)
    out_wait(1)


_gather_sum = functools.partial(
    pl.kernel,
    out_type=jax.ShapeDtypeStruct((B, EMB), jnp.float32),
    mesh=plsc.VectorSubcoreMesh(core_axis_name="c", subcore_axis_name="s"),
    scratch_types=[
        pltpu.VMEM((PB, L), jnp.int32),
        pltpu.VMEM((PB, L), jnp.int32),
        pltpu.VMEM((PB * L, EMB), jnp.float32),
        pltpu.VMEM((PB * L, EMB), jnp.float32),
        pltpu.VMEM((PB, EMB), jnp.float32),
        pltpu.VMEM((PB, EMB), jnp.float32),
        pltpu.SemaphoreType.DMA,
        pltpu.SemaphoreType.DMA,
        pltpu.SemaphoreType.DMA,
        pltpu.SemaphoreType.DMA,
        pltpu.SemaphoreType.DMA,
        pltpu.SemaphoreType.DMA,
    ],
    compiler_params=pltpu.CompilerParams(
        use_tc_tiling_on_sc=False, needs_layout_passes=False),
)(_gather_sum_kernel)


BLK = 2048


def _dense_body(h_ref, lab_ref, w1_ref, b1_ref, w2_ref, b2_ref,
                l1_ref, leaf_ref):
    h = h_ref[...]
    l1_ref[...] = (
        jnp.dot(h, w1_ref[...], preferred_element_type=jnp.float32)
        + b1_ref[...]
    )
    one_hot = (
        lab_ref[...]
        == lax.broadcasted_iota(jnp.int32, (BLK, NUM_L1), 1)
    ).astype(jnp.float32)
    leaf_ref[...] = (
        jnp.dot(h, w2_ref[0:EMB, :], preferred_element_type=jnp.float32)
        + jnp.dot(one_hot, w2_ref[EMB:, :], preferred_element_type=jnp.float32)
        + b2_ref[...]
    )


def kernel(x, level1_labels, emb, W1, b1, W2, b2):
    embr = lax.optimization_barrier(emb.reshape(VOCAB // 2, 2 * EMB))
    h = _gather_sum(x, embr.reshape(VOCAB, EMB))

    lab2d = level1_labels.reshape(B, 1)
    grid = B // BLK
    l1, leaf = pl.pallas_call(
        _dense_body,
        grid=(grid,),
        in_specs=[
            pl.BlockSpec((BLK, EMB), lambda i: (i, 0)),
            pl.BlockSpec((BLK, 1), lambda i: (i, 0)),
            pl.BlockSpec((EMB, NUM_L1), lambda i: (0, 0)),
            pl.BlockSpec((1, NUM_L1), lambda i: (0, 0)),
            pl.BlockSpec((EMB + NUM_L1, NUM_LEAF), lambda i: (0, 0)),
            pl.BlockSpec((1, NUM_LEAF), lambda i: (0, 0)),
        ],
        out_specs=[
            pl.BlockSpec((BLK, NUM_L1), lambda i: (i, 0)),
            pl.BlockSpec((BLK, NUM_LEAF), lambda i: (i, 0)),
        ],
        out_shape=[
            jax.ShapeDtypeStruct((B, NUM_L1), jnp.float32),
            jax.ShapeDtypeStruct((B, NUM_LEAF), jnp.float32),
        ],
    )(h, lab2d, W1, b1.reshape(1, NUM_L1), W2, b2.reshape(1, NUM_LEAF))
    return (l1, leaf)


# speedup vs baseline: 1.3132x; 1.0017x over previous
import functools

import jax
import jax.numpy as jnp
from jax import lax
from jax.experimental import pallas as pl
from jax.experimental.pallas import tpu as pltpu
from jax.experimental.pallas import tpu_sc as plsc

VOCAB = 1000000
EMB = 64
NUM_L1 = 32
NUM_LEAF = 1024
B = 16384
L = 200
SPLIT = 104

NC, NS = 2, 16
NW = NC * NS
ROWS_PER_W = B // NW
PB = 4
N_STEP = ROWS_PER_W // PB
LANES = 16
C_CHUNKS = EMB // LANES
UNROLL = 8


def _gather_sum_kernel(x_hbm, emb_hbm, out_hbm,
                       idx0, idx1, rows0, rows1, os0, os1,
                       isem0, isem1, gsem0, gsem1, osem0, osem1):
    idx_v = (idx0, idx1)
    rows_v = (rows0, rows1)
    out_v = (os0, os1)
    isem = (isem0, isem1)
    gsem = (gsem0, gsem1)
    osem = (osem0, osem1)
    wid = lax.axis_index("s") * NC + lax.axis_index("c")
    base = wid * ROWS_PER_W

    def idx_start(slot, step):
        pltpu.async_copy(
            x_hbm.at[pl.ds(base + step * PB, PB)], idx_v[slot], isem[slot])

    def idx_wait(slot):
        pltpu.make_async_copy(
            x_hbm.at[pl.ds(0, PB)], idx_v[slot], isem[slot]).wait()

    def gathers_start(slot):
        for k in range(PB):
            pltpu.async_copy(
                emb_hbm.at[idx_v[slot].at[k, pl.ds(0, SPLIT)]],
                rows_v[slot].at[pl.ds(k * L, SPLIT)],
                gsem[slot])
            pltpu.async_copy(
                emb_hbm.at[idx_v[slot].at[k, pl.ds(SPLIT, L - SPLIT)]],
                rows_v[slot].at[pl.ds(k * L + SPLIT, L - SPLIT)],
                gsem[slot])

    def gathers_wait(slot):
        pltpu.make_async_copy(
            emb_hbm.at[pl.ds(0, PB * L)], rows_v[slot], gsem[slot]).wait()

    def out_start(slot, step):
        pltpu.async_copy(
            out_v[slot], out_hbm.at[pl.ds(base + step * PB, PB)], osem[slot])

    def out_wait(slot):
        pltpu.make_async_copy(
            out_v[slot], out_hbm.at[pl.ds(0, PB)], osem[slot]).wait()

    def count_recips(slot):
        lane = lax.iota(jnp.int32, 16)
        recips = []
        for k in range(PB):
            total = jnp.zeros((LANES,), jnp.int32)
            for c in range(12):
                ch = idx_v[slot][k, pl.ds(c * LANES, LANES)]
                total = total + plsc.all_reduce_population_count(ch != 0)
            tail = idx_v[slot][k, pl.ds(L - LANES, LANES)]
            total = total + plsc.all_reduce_population_count(
                (tail != 0) & (lane >= 12 * LANES - (L - LANES)))
            recips.append(1.0 / total.astype(jnp.float32))
        return recips

    def reduce(slot, recips):
        zero = jnp.zeros((LANES,), jnp.float32)
        for k in range(PB):
            rbase = k * L

            def body(r, acc, rbase=rbase, slot=slot):
                a = list(acc)
                for u in range(UNROLL):
                    rr = rbase + r * UNROLL + u
                    for c in range(C_CHUNKS):
                        a[c] = a[c] + rows_v[slot][rr, pl.ds(c * LANES, LANES)]
                return tuple(a)

            acc = lax.fori_loop(0, L // UNROLL, body, (zero,) * C_CHUNKS)
            for c in range(C_CHUNKS):
                out_v[slot][k, pl.ds(c * LANES, LANES)] = (
                    jnp.maximum(acc[c] * recips[k], 0.0))

    idx_start(0, 0)
    idx_wait(0)
    gathers_start(0)
    idx_start(1, 1)

    def body(i2, _):
        s0 = 2 * i2
        s1 = s0 + 1
        idx_wait(1)
        gathers_start(1)
        gathers_wait(0)
        r0 = count_recips(0)

        @pl.when(s0 + 2 < N_STEP)
        def _():
            idx_start(0, s0 + 2)

        @pl.when(s0 >= 2)
        def _():
            out_wait(0)

        reduce(0, r0)
        out_start(0, s0)

        @pl.when(s0 + 2 < N_STEP)
        def _():
            idx_wait(0)
            gathers_start(0)

        gathers_wait(1)
        r1 = count_recips(1)

        @pl.when(s1 + 2 < N_STEP)
        def _():
            idx_start(1, s1 + 2)

        @pl.when(s1 >= 2)
        def _():
            out_wait(1)

        reduce(1, r1)
        out_start(1, s1)
        return 0

    lax.fori_loop(0, N_STEP // 2, body, 0)
    out_wait(0)
    out_wait(1)


_gather_sum = functools.partial(
    pl.kernel,
    out_type=jax.ShapeDtypeStruct((B, EMB), jnp.float32),
    mesh=plsc.VectorSubcoreMesh(core_axis_name="c", subcore_axis_name="s"),
    scratch_types=[
        pltpu.VMEM((PB, L), jnp.int32),
        pltpu.VMEM((PB, L), jnp.int32),
        pltpu.VMEM((PB * L, EMB), jnp.float32),
        pltpu.VMEM((PB * L, EMB), jnp.float32),
        pltpu.VMEM((PB, EMB), jnp.float32),
        pltpu.VMEM((PB, EMB), jnp.float32),
        pltpu.SemaphoreType.DMA,
        pltpu.SemaphoreType.DMA,
        pltpu.SemaphoreType.DMA,
        pltpu.SemaphoreType.DMA,
        pltpu.SemaphoreType.DMA,
        pltpu.SemaphoreType.DMA,
    ],
    compiler_params=pltpu.CompilerParams(
        use_tc_tiling_on_sc=False, needs_layout_passes=False),
)(_gather_sum_kernel)


BLK = 2048


def _dense_body(h_ref, lab_ref, w1_ref, b1_ref, w2_ref, b2_ref,
                l1_ref, leaf_ref):
    h = h_ref[...]
    l1_ref[...] = (
        jnp.dot(h, w1_ref[...], preferred_element_type=jnp.float32)
        + b1_ref[...]
    )
    one_hot = (
        lab_ref[...]
        == lax.broadcasted_iota(jnp.int32, (BLK, NUM_L1), 1)
    ).astype(jnp.float32)
    leaf_ref[...] = (
        jnp.dot(h, w2_ref[0:EMB, :], preferred_element_type=jnp.float32)
        + jnp.dot(one_hot, w2_ref[EMB:, :], preferred_element_type=jnp.float32)
        + b2_ref[...]
    )


def kernel(x, level1_labels, emb, W1, b1, W2, b2):
    h = _gather_sum(x, emb)

    lab2d = level1_labels.reshape(B, 1)
    grid = B // BLK
    l1, leaf = pl.pallas_call(
        _dense_body,
        grid=(grid,),
        in_specs=[
            pl.BlockSpec((BLK, EMB), lambda i: (i, 0)),
            pl.BlockSpec((BLK, 1), lambda i: (i, 0)),
            pl.BlockSpec((EMB, NUM_L1), lambda i: (0, 0)),
            pl.BlockSpec((1, NUM_L1), lambda i: (0, 0)),
            pl.BlockSpec((EMB + NUM_L1, NUM_LEAF), lambda i: (0, 0)),
            pl.BlockSpec((1, NUM_LEAF), lambda i: (0, 0)),
        ],
        out_specs=[
            pl.BlockSpec((BLK, NUM_L1), lambda i: (i, 0)),
            pl.BlockSpec((BLK, NUM_LEAF), lambda i: (i, 0)),
        ],
        out_shape=[
            jax.ShapeDtypeStruct((B, NUM_L1), jnp.float32),
            jax.ShapeDtypeStruct((B, NUM_LEAF), jnp.float32),
        ],
    )(h, lab2d, W1, b1.reshape(1, NUM_L1), W2, b2.reshape(1, NUM_LEAF))
    return (l1, leaf)


# speedup vs baseline: 1.3139x; 1.0006x over previous
import functools

import jax
import jax.numpy as jnp
from jax import lax
from jax.experimental import pallas as pl
from jax.experimental.pallas import tpu as pltpu
from jax.experimental.pallas import tpu_sc as plsc

VOCAB = 1000000
EMB = 64
NUM_L1 = 32
NUM_LEAF = 1024
B = 16384
L = 200
SPLIT = 104

NC, NS = 2, 16
NW = NC * NS
ROWS_PER_W = B // NW
PB = 4
N_STEP = ROWS_PER_W // PB
LANES = 16
C_CHUNKS = EMB // LANES
UNROLL = 10


def _gather_sum_kernel(x_hbm, emb_hbm, out_hbm,
                       idx0, idx1, rows0, rows1, os0, os1,
                       isem0, isem1, gsem0, gsem1, osem0, osem1):
    idx_v = (idx0, idx1)
    rows_v = (rows0, rows1)
    out_v = (os0, os1)
    isem = (isem0, isem1)
    gsem = (gsem0, gsem1)
    osem = (osem0, osem1)
    wid = lax.axis_index("s") * NC + lax.axis_index("c")
    base = wid * ROWS_PER_W

    def idx_start(slot, step):
        pltpu.async_copy(
            x_hbm.at[pl.ds(base + step * PB, PB)], idx_v[slot], isem[slot])

    def idx_wait(slot):
        pltpu.make_async_copy(
            x_hbm.at[pl.ds(0, PB)], idx_v[slot], isem[slot]).wait()

    def gathers_start(slot):
        for k in range(PB):
            pltpu.async_copy(
                emb_hbm.at[idx_v[slot].at[k, pl.ds(0, SPLIT)]],
                rows_v[slot].at[pl.ds(k * L, SPLIT)],
                gsem[slot])
            pltpu.async_copy(
                emb_hbm.at[idx_v[slot].at[k, pl.ds(SPLIT, L - SPLIT)]],
                rows_v[slot].at[pl.ds(k * L + SPLIT, L - SPLIT)],
                gsem[slot])

    def gathers_wait(slot):
        pltpu.make_async_copy(
            emb_hbm.at[pl.ds(0, PB * L)], rows_v[slot], gsem[slot]).wait()

    def out_start(slot, step):
        pltpu.async_copy(
            out_v[slot], out_hbm.at[pl.ds(base + step * PB, PB)], osem[slot])

    def out_wait(slot):
        pltpu.make_async_copy(
            out_v[slot], out_hbm.at[pl.ds(0, PB)], osem[slot]).wait()

    def count_recips(slot):
        lane = lax.iota(jnp.int32, 16)
        recips = []
        for k in range(PB):
            total = jnp.zeros((LANES,), jnp.int32)
            for c in range(12):
                ch = idx_v[slot][k, pl.ds(c * LANES, LANES)]
                total = total + plsc.all_reduce_population_count(ch != 0)
            tail = idx_v[slot][k, pl.ds(L - LANES, LANES)]
            total = total + plsc.all_reduce_population_count(
                (tail != 0) & (lane >= 12 * LANES - (L - LANES)))
            recips.append(1.0 / total.astype(jnp.float32))
        return recips

    def reduce(slot, recips):
        zero = jnp.zeros((LANES,), jnp.float32)
        for k in range(PB):
            rbase = k * L

            def body(r, acc, rbase=rbase, slot=slot):
                a = list(acc)
                for u in range(UNROLL):
                    rr = rbase + r * UNROLL + u
                    for c in range(C_CHUNKS):
                        a[c] = a[c] + rows_v[slot][rr, pl.ds(c * LANES, LANES)]
                return tuple(a)

            acc = lax.fori_loop(0, L // UNROLL, body, (zero,) * C_CHUNKS)
            for c in range(C_CHUNKS):
                out_v[slot][k, pl.ds(c * LANES, LANES)] = (
                    jnp.maximum(acc[c] * recips[k], 0.0))

    idx_start(0, 0)
    idx_wait(0)
    gathers_start(0)
    idx_start(1, 1)

    def body(i2, _):
        s0 = 2 * i2
        s1 = s0 + 1
        idx_wait(1)
        gathers_start(1)
        gathers_wait(0)
        r0 = count_recips(0)

        @pl.when(s0 + 2 < N_STEP)
        def _():
            idx_start(0, s0 + 2)

        @pl.when(s0 >= 2)
        def _():
            out_wait(0)

        reduce(0, r0)
        out_start(0, s0)

        @pl.when(s0 + 2 < N_STEP)
        def _():
            idx_wait(0)
            gathers_start(0)

        gathers_wait(1)
        r1 = count_recips(1)

        @pl.when(s1 + 2 < N_STEP)
        def _():
            idx_start(1, s1 + 2)

        @pl.when(s1 >= 2)
        def _():
            out_wait(1)

        reduce(1, r1)
        out_start(1, s1)
        return 0

    lax.fori_loop(0, N_STEP // 2, body, 0)
    out_wait(0)
    out_wait(1)


_gather_sum = functools.partial(
    pl.kernel,
    out_type=jax.ShapeDtypeStruct((B, EMB), jnp.float32),
    mesh=plsc.VectorSubcoreMesh(core_axis_name="c", subcore_axis_name="s"),
    scratch_types=[
        pltpu.VMEM((PB, L), jnp.int32),
        pltpu.VMEM((PB, L), jnp.int32),
        pltpu.VMEM((PB * L, EMB), jnp.float32),
        pltpu.VMEM((PB * L, EMB), jnp.float32),
        pltpu.VMEM((PB, EMB), jnp.float32),
        pltpu.VMEM((PB, EMB), jnp.float32),
        pltpu.SemaphoreType.DMA,
        pltpu.SemaphoreType.DMA,
        pltpu.SemaphoreType.DMA,
        pltpu.SemaphoreType.DMA,
        pltpu.SemaphoreType.DMA,
        pltpu.SemaphoreType.DMA,
    ],
    compiler_params=pltpu.CompilerParams(
        use_tc_tiling_on_sc=False, needs_layout_passes=False),
)(_gather_sum_kernel)


BLK = 2048


def _dense_body(h_ref, lab_ref, w1_ref, b1_ref, w2_ref, b2_ref,
                l1_ref, leaf_ref):
    h = h_ref[...]
    l1_ref[...] = (
        jnp.dot(h, w1_ref[...], preferred_element_type=jnp.float32)
        + b1_ref[...]
    )
    one_hot = (
        lab_ref[...]
        == lax.broadcasted_iota(jnp.int32, (BLK, NUM_L1), 1)
    ).astype(jnp.float32)
    leaf_ref[...] = (
        jnp.dot(h, w2_ref[0:EMB, :], preferred_element_type=jnp.float32)
        + jnp.dot(one_hot, w2_ref[EMB:, :], preferred_element_type=jnp.float32)
        + b2_ref[...]
    )


def kernel(x, level1_labels, emb, W1, b1, W2, b2):
    h = _gather_sum(x, emb)

    lab2d = level1_labels.reshape(B, 1)
    grid = B // BLK
    l1, leaf = pl.pallas_call(
        _dense_body,
        grid=(grid,),
        in_specs=[
            pl.BlockSpec((BLK, EMB), lambda i: (i, 0)),
            pl.BlockSpec((BLK, 1), lambda i: (i, 0)),
            pl.BlockSpec((EMB, NUM_L1), lambda i: (0, 0)),
            pl.BlockSpec((1, NUM_L1), lambda i: (0, 0)),
            pl.BlockSpec((EMB + NUM_L1, NUM_LEAF), lambda i: (0, 0)),
            pl.BlockSpec((1, NUM_LEAF), lambda i: (0, 0)),
        ],
        out_specs=[
            pl.BlockSpec((BLK, NUM_L1), lambda i: (i, 0)),
            pl.BlockSpec((BLK, NUM_LEAF), lambda i: (i, 0)),
        ],
        out_shape=[
            jax.ShapeDtypeStruct((B, NUM_L1), jnp.float32),
            jax.ShapeDtypeStruct((B, NUM_LEAF), jnp.float32),
        ],
    )(h, lab2d, W1, b1.reshape(1, NUM_L1), W2, b2.reshape(1, NUM_LEAF))
    return (l1, leaf)
